# baseline (speedup 1.0000x reference)
import functools

import jax
import jax.numpy as jnp
from jax import lax
from jax.experimental import pallas as pl
from jax.experimental.pallas import tpu as pltpu
from jax.experimental.pallas import tpu_sc as plsc

N = 10000
E = 160000
D = 128
D2 = 256
EA = 16
NC = 2
NS = 16
NW = NC * NS
EPAD = 163840
EPT = EPAD // NW
CG = 64
QA = 104
QB = 56
IPAD = NS * CG * QA + NS * CG * QB + (QA - QB) * CG
CS = 128
NCHS = EPT // CS
NPAD = 10240
N8 = NPAD // 8
BE = 2048
BN = 2000
BNP4 = BN // 4

f32 = jnp.float32


def _silu(v):
    return v * (1.0 / (1.0 + jnp.exp(-v)))


def _make_sc_gather():
    mesh = plsc.VectorSubcoreMesh(core_axis_name="c", subcore_axis_name="s")
    out_type = [
        jax.ShapeDtypeStruct((EPAD, D2), f32),
        jax.ShapeDtypeStruct((EPAD, D2), f32),
    ]
    scratch = [
        pltpu.VMEM((QA * CG,), jnp.int32),
        pltpu.VMEM((QA * CG,), jnp.int32),
        pltpu.VMEM((2, CG, D2), f32),
        pltpu.VMEM((2, CG, D2), f32),
        pltpu.SemaphoreType.DMA,
        pltpu.SemaphoreType.DMA,
    ]

    @functools.partial(pl.kernel, mesh=mesh, out_type=out_type,
                       scratch_types=scratch)
    def gather_k(ac_hbm, bc_hbm, row_hbm, col_hbm,
                 ar_hbm, bc_out_hbm,
                 idxr, idxc, buf0, buf1, gsem, osem):
        cid = lax.axis_index("c")
        sid = lax.axis_index("s")
        nch = lax.select(cid == 0, QA, QB)
        tbase = lax.select(cid == 0, sid * (QA * CG),
                           NS * (QA * CG) + sid * (QB * CG))
        tbase = pl.multiple_of(tbase, CG)
        pltpu.sync_copy(row_hbm.at[pl.ds(tbase, QA * CG)], idxr)
        pltpu.sync_copy(col_hbm.at[pl.ds(tbase, QA * CG)], idxc)

        def fire_gather(j, p):
            pltpu.async_copy(ac_hbm.at[idxr.at[pl.ds(j * CG, CG)]],
                             buf0.at[p], gsem)
            pltpu.async_copy(bc_hbm.at[idxc.at[pl.ds(j * CG, CG)]],
                             buf1.at[p], gsem)

        def wait_gather(j, p):
            pltpu.make_async_copy(ac_hbm.at[idxr.at[pl.ds(j * CG, CG)]],
                                  buf0.at[p], gsem).wait()
            pltpu.make_async_copy(bc_hbm.at[idxc.at[pl.ds(j * CG, CG)]],
                                  buf1.at[p], gsem).wait()

        def fire_out(j, p):
            pltpu.async_copy(buf0.at[p], ar_hbm.at[pl.ds(tbase + j * CG, CG)],
                             osem)
            pltpu.async_copy(buf1.at[p],
                             bc_out_hbm.at[pl.ds(tbase + j * CG, CG)], osem)

        def wait_out(j, p):
            pltpu.make_async_copy(buf0.at[p],
                                  ar_hbm.at[pl.ds(tbase + j * CG, CG)],
                                  osem).wait()
            pltpu.make_async_copy(buf1.at[p],
                                  bc_out_hbm.at[pl.ds(tbase + j * CG, CG)],
                                  osem).wait()

        fire_gather(0, 0)

        def body(j, carry):
            cur = lax.rem(j, 2)
            oth = 1 - cur

            @pl.when(j > 0)
            def _():
                wait_out(j - 1, oth)

            @pl.when(j < nch - 1)
            def _():
                fire_gather(j + 1, oth)

            wait_gather(j, cur)
            fire_out(j, cur)
            return carry

        lax.fori_loop(0, nch, body, 0)
        wait_out(nch - 1, 1)

    return gather_k


def _make_sc_scatter(nacc):
    mesh = plsc.VectorSubcoreMesh(core_axis_name="c", subcore_axis_name="s")
    out_type = jax.ShapeDtypeStruct((NC, nacc, D), f32)
    scratch = [
        pltpu.VMEM((NCHS, CS), jnp.int32),
        pltpu.VMEM((2, CS, D), f32),
        pltpu.VMEM_SHARED((nacc, D), f32),
        pltpu.SemaphoreType.DMA,
        pltpu.SemaphoreType.DMA,
    ]
    RPT = nacc // NS

    @functools.partial(pl.kernel, mesh=mesh, out_type=out_type,
                       scratch_types=scratch)
    def scatter_k(ef_hbm, idx2_hbm, zm_hbm, pm_hbm,
                  idxs, bufe, accm, rsem, ssem):
        cid = lax.axis_index("c")
        sid = lax.axis_index("s")
        wid = sid * NC + cid
        pltpu.sync_copy(idx2_hbm.at[pl.ds(wid * NCHS, NCHS)], idxs)
        pltpu.sync_copy(zm_hbm.at[pl.ds(sid * RPT, RPT)],
                        accm.at[pl.ds(sid * RPT, RPT)])
        plsc.subcore_barrier()

        def fire_read(j, p):
            base = wid * EPT + j * CS
            pltpu.async_copy(ef_hbm.at[pl.ds(base, CS)], bufe.at[p], rsem)

        def wait_read(j, p):
            base = wid * EPT + j * CS
            pltpu.make_async_copy(ef_hbm.at[pl.ds(base, CS)], bufe.at[p],
                                  rsem).wait()

        def fire_add(j, p):
            pltpu.async_copy(bufe.at[p], accm.at[idxs.at[j]], ssem, add=True)

        def wait_add(j, p):
            pltpu.make_async_copy(bufe.at[p], accm.at[idxs.at[j]], ssem).wait()

        fire_read(0, 0)

        def body(j, carry):
            cur = lax.rem(j, 2)
            oth = 1 - cur

            @pl.when(j > 0)
            def _():
                wait_add(j - 1, oth)

            @pl.when(j < NCHS - 1)
            def _():
                fire_read(j + 1, oth)

            wait_read(j, cur)
            fire_add(j, cur)
            return carry

        lax.fori_loop(0, NCHS, body, 0)
        wait_add(NCHS - 1, (NCHS - 1) % 2)
        plsc.subcore_barrier()
        pltpu.sync_copy(accm.at[pl.ds(sid * RPT, RPT)],
                        pm_hbm.at[cid, pl.ds(sid * RPT, RPT)])

    return scatter_k


_SC_GATHER = None
_SC_SCATTER_N = None
_SC_SCATTER_8 = None


def _sc_gather(ac, bc, rowg, colg):
    global _SC_GATHER
    if _SC_GATHER is None:
        _SC_GATHER = _make_sc_gather()
    return _SC_GATHER(ac, bc, rowg, colg)


def _sc_scatter_n(ef, row2, zm):
    global _SC_SCATTER_N
    if _SC_SCATTER_N is None:
        _SC_SCATTER_N = _make_sc_scatter(NPAD)
    return _SC_SCATTER_N(ef, row2, zm)


def _sc_scatter_8(sm, row82, zm):
    global _SC_SCATTER_8
    if _SC_SCATTER_8 is None:
        _SC_SCATTER_8 = _make_sc_scatter(N8)
    return _SC_SCATTER_8(sm, row82, zm)


def _tc_linear(x, wT, bias):
    nb = N // BN

    def body(x_r, w_r, b_r, o_r):
        o_r[...] = jnp.dot(x_r[...], w_r[...],
                           preferred_element_type=f32) + b_r[...]

    return pl.pallas_call(
        body,
        grid=(nb,),
        in_specs=[
            pl.BlockSpec((BN, D), lambda p: (p, 0)),
            pl.BlockSpec((D, D), lambda p: (0, 0)),
            pl.BlockSpec((1, D), lambda p: (0, 0)),
        ],
        out_specs=pl.BlockSpec((BN, D), lambda p: (p, 0)),
        out_shape=jax.ShapeDtypeStruct((N, D), f32),
    )(x, wT, bias)


def _tc_pre(h, coord, waT, b1, wbT):
    nb = N // BN

    def body(h_r, c_r, wa_r, b1_r, wb_r, ac_r, bc_r):
        hv = h_r[...]
        cv = c_r[...]
        ac_r[:, :D] = jnp.dot(hv, wa_r[...], preferred_element_type=f32) + b1_r[...]
        ac_r[:, D:] = cv
        bc_r[:, :D] = jnp.dot(hv, wb_r[...], preferred_element_type=f32)
        bc_r[:, D:] = cv

    return pl.pallas_call(
        body,
        grid=(nb,),
        in_specs=[
            pl.BlockSpec((BN, D), lambda p: (p, 0)),
            pl.BlockSpec((BN, D), lambda p: (p, 0)),
            pl.BlockSpec((D, D), lambda p: (0, 0)),
            pl.BlockSpec((1, D), lambda p: (0, 0)),
            pl.BlockSpec((D, D), lambda p: (0, 0)),
        ],
        out_specs=[
            pl.BlockSpec((BN, D2), lambda p: (p, 0)),
            pl.BlockSpec((BN, D2), lambda p: (p, 0)),
        ],
        out_shape=[
            jax.ShapeDtypeStruct((N, D2), f32),
            jax.ShapeDtypeStruct((N, D2), f32),
        ],
    )(h, coord, waT, b1, wbT)


def _tc_edge(acr, bcc, eap, rowe, w1dT, w1c, w2T, b2, w3T, b3, w4):
    nb = EPAD // BE

    def body(ac_r, bc_r, ea_r, row_r,
             w1d_r, w1c_r, w2_r, b2_r, w3_r, b3_r, w4_r,
             ef_o, sm_o):
        p = pl.program_id(0)
        acv = ac_r[...]
        bcv = bc_r[...]
        ar = acv[:, :D]
        cr = acv[:, D:]
        br = bcv[:, :D]
        cc = bcv[:, D:]
        cd = cr - cc
        radial = jnp.sum(cd * cd, axis=1, keepdims=True)
        pre = (ar + br + radial * w1c_r[...]
               + jnp.dot(ea_r[...], w1d_r[...], preferred_element_type=f32))
        m = _silu(pre)
        ef = _silu(jnp.dot(m, w2_r[...], preferred_element_type=f32) + b2_r[...])
        t = _silu(jnp.dot(ef, w3_r[...], preferred_element_type=f32) + b3_r[...])
        w = jnp.sum(t * w4_r[...], axis=1, keepdims=True)
        rowv = row_r[...]
        base_l = 16 * lax.rem(rowv, 8)
        lane = lax.broadcasted_iota(jnp.int32, (BE, D), 1)
        tx = cd[:, 0:1] * w
        ty = cd[:, 1:2] * w
        tz = cd[:, 2:3] * w
        sm = (tx * (lane == base_l) + ty * (lane == base_l + 1)
              + tz * (lane == base_l + 2) + (lane == base_l + 3).astype(f32))
        rowid = p * BE + lax.broadcasted_iota(jnp.int32, (BE, 1), 0)
        maskf = (rowid < E).astype(f32)
        ef_o[...] = ef * maskf
        sm_o[...] = sm * maskf

    return pl.pallas_call(
        body,
        grid=(nb,),
        in_specs=[
            pl.BlockSpec((BE, D2), lambda p: (p, 0)),
            pl.BlockSpec((BE, D2), lambda p: (p, 0)),
            pl.BlockSpec((BE, EA), lambda p: (p, 0)),
            pl.BlockSpec((BE, 1), lambda p: (p, 0)),
            pl.BlockSpec((EA, D), lambda p: (0, 0)),
            pl.BlockSpec((1, D), lambda p: (0, 0)),
            pl.BlockSpec((D, D), lambda p: (0, 0)),
            pl.BlockSpec((1, D), lambda p: (0, 0)),
            pl.BlockSpec((D, D), lambda p: (0, 0)),
            pl.BlockSpec((1, D), lambda p: (0, 0)),
            pl.BlockSpec((1, D), lambda p: (0, 0)),
        ],
        out_specs=[
            pl.BlockSpec((BE, D), lambda p: (p, 0)),
            pl.BlockSpec((BE, D), lambda p: (p, 0)),
        ],
        out_shape=[
            jax.ShapeDtypeStruct((EPAD, D), f32),
            jax.ShapeDtypeStruct((EPAD, D), f32),
        ],
    )(acr, bcc, eap, rowe, w1dT, w1c, w2T, b2, w3T, b3, w4)


def _tc_node(h, coord, pm, ps, wn1aT, wn1bT, bn1, wn2T, bn2):
    BNN = 2048
    BNP8N = BNN // 8
    nb = NPAD // BNN

    def body(h_r, c_r, pm_r, ps_r, wa_r, wb_r, b1_r, w2_r, b2_r,
             ho_r, co_r):
        magg = pm_r[0] + pm_r[1]
        packed = ps_r[0] + ps_r[1]
        ri = lax.broadcasted_iota(jnp.int32, (BNN, BNP8N), 0)
        ci = lax.broadcasted_iota(jnp.int32, (BNN, BNP8N), 1)
        pmat = ((ri // 8) == ci).astype(f32)
        rows_exp = jnp.dot(pmat, packed, preferred_element_type=f32)
        m8 = lax.rem(lax.broadcasted_iota(jnp.int32, (BNN, 1), 0), 8)
        base_l = 16 * m8
        lane = lax.broadcasted_iota(jnp.int32, (BNN, D), 1)
        tx = jnp.sum(jnp.where(lane == base_l, rows_exp, 0.0), axis=1,
                     keepdims=True)
        ty = jnp.sum(jnp.where(lane == base_l + 1, rows_exp, 0.0), axis=1,
                     keepdims=True)
        tz = jnp.sum(jnp.where(lane == base_l + 2, rows_exp, 0.0), axis=1,
                     keepdims=True)
        cnt = jnp.sum(jnp.where(lane == base_l + 3, rows_exp, 0.0), axis=1,
                      keepdims=True)
        agg = (tx * (lane == 0) + ty * (lane == 1) + tz * (lane == 2))
        co_r[...] = c_r[...] + agg / jnp.maximum(cnt, 1.0)
        hv = h_r[...]
        hh = _silu(jnp.dot(hv, wa_r[...], preferred_element_type=f32)
                   + jnp.dot(magg, wb_r[...], preferred_element_type=f32)
                   + b1_r[...])
        ho_r[...] = hv + jnp.dot(hh, w2_r[...], preferred_element_type=f32) + b2_r[...]

    return pl.pallas_call(
        body,
        grid=(nb,),
        in_specs=[
            pl.BlockSpec((BNN, D), lambda p: (p, 0)),
            pl.BlockSpec((BNN, D), lambda p: (p, 0)),
            pl.BlockSpec((NC, BNN, D), lambda p: (0, p, 0)),
            pl.BlockSpec((NC, BNP8N, D), lambda p: (0, p, 0)),
            pl.BlockSpec((D, D), lambda p: (0, 0)),
            pl.BlockSpec((D, D), lambda p: (0, 0)),
            pl.BlockSpec((1, D), lambda p: (0, 0)),
            pl.BlockSpec((D, D), lambda p: (0, 0)),
            pl.BlockSpec((1, D), lambda p: (0, 0)),
        ],
        out_specs=[
            pl.BlockSpec((BNN, D), lambda p: (p, 0)),
            pl.BlockSpec((BNN, D), lambda p: (p, 0)),
        ],
        out_shape=[
            jax.ShapeDtypeStruct((N, D), f32),
            jax.ShapeDtypeStruct((N, D), f32),
        ],
    )(h, coord, pm, ps, wn1aT, wn1bT, bn1, wn2T, bn2)


def _tc_mean(h):
    nb = N // BN

    def body(h_r, o_r):
        p = pl.program_id(0)
        part = jnp.sum(h_r[...], axis=0, keepdims=True) * (1.0 / N)

        @pl.when(p == 0)
        def _():
            o_r[...] = part

        @pl.when(p != 0)
        def _():
            o_r[...] = o_r[...] + part

    return pl.pallas_call(
        body,
        grid=(nb,),
        in_specs=[pl.BlockSpec((BN, D), lambda p: (p, 0))],
        out_specs=pl.BlockSpec((1, D), lambda p: (0, 0)),
        out_shape=jax.ShapeDtypeStruct((1, D), f32),
    )(h)


def kernel(h, x, edges, edge_attr, params):
    row = edges[0].astype(jnp.int32)
    col = edges[1].astype(jnp.int32)
    pad = EPAD - E
    rowp = jnp.concatenate([row, jnp.zeros((pad,), jnp.int32)])
    colp = jnp.concatenate([col, jnp.zeros((pad,), jnp.int32)])
    rowg = jnp.concatenate([rowp, jnp.zeros((IPAD - EPAD,), jnp.int32)])
    colg = jnp.concatenate([colp, jnp.zeros((IPAD - EPAD,), jnp.int32)])
    rowe = rowp.reshape(EPAD, 1)
    row2 = rowp.reshape(EPAD // CS, CS)
    row82 = (rowp // 8).reshape(EPAD // CS, CS)
    eap = jnp.concatenate([edge_attr, jnp.zeros((pad, EA), f32)], axis=0)
    coord = jnp.concatenate([x, jnp.zeros((N, D - 3), f32)], axis=1)
    zm = jnp.zeros((NPAD, D), f32)

    for bp in params:
        wi = bp["emb_in"]
        h = _tc_linear(h, wi["W"].T, wi["b"][None, :])
        for gp in bp["gcls"]:
            w1 = gp["edge_mlp"][0]["W"]
            b1 = gp["edge_mlp"][0]["b"]
            w2 = gp["edge_mlp"][1]["W"]
            b2 = gp["edge_mlp"][1]["b"]
            w3 = gp["coord_mlp"][0]["W"]
            b3 = gp["coord_mlp"][0]["b"]
            w4 = gp["coord_mlp"][1]["W"]
            wn1 = gp["node_mlp"][0]["W"]
            bn1 = gp["node_mlp"][0]["b"]
            wn2 = gp["node_mlp"][1]["W"]
            bn2 = gp["node_mlp"][1]["b"]

            ac, bc = _tc_pre(h, coord, w1[:, :D].T, b1[None, :],
                             w1[:, D:2 * D].T)
            acr, bcc = _sc_gather(ac, bc, rowg, colg)
            ef, sm = _tc_edge(acr, bcc, eap, rowe,
                              w1[:, 2 * D + 1:].T, w1[:, 2 * D][None, :],
                              w2.T, b2[None, :], w3.T, b3[None, :], w4)
            pm = _sc_scatter_n(ef, row2, zm)
            ps = _sc_scatter_8(sm, row82, zm)
            h, coord = _tc_node(h, coord, pm, ps,
                                wn1[:, :D].T, wn1[:, D:].T, bn1[None, :],
                                wn2.T, bn2[None, :])
        wo = bp["emb_out"]
        h = _tc_linear(h, wo["W"].T, wo["b"][None, :])

    mol = _tc_mean(h)
    return (mol, h, coord[:, :3])


# speedup vs baseline: 1.0194x; 1.0194x over previous
import functools

import jax
import jax.numpy as jnp
from jax import lax
from jax.experimental import pallas as pl
from jax.experimental.pallas import tpu as pltpu
from jax.experimental.pallas import tpu_sc as plsc

N = 10000
E = 160000
D = 128
D2 = 256
EA = 16
NC = 2
NS = 16
NW = NC * NS
EPAD = 163840
EPT = EPAD // NW
CG = 64
QA = 56
QB = 104
QM = max(QA, QB)
IPAD = EPAD + (QM - QB) * CG
CS = 128
NCHS = EPT // CS
NPAD = 10240
N8 = NPAD // 8
BE = 2048
BN = 2000
BNP4 = BN // 4

f32 = jnp.float32


def _silu(v):
    return v * (1.0 / (1.0 + jnp.exp(-v)))


def _make_sc_gather():
    mesh = plsc.VectorSubcoreMesh(core_axis_name="c", subcore_axis_name="s")
    out_type = [
        jax.ShapeDtypeStruct((EPAD, D2), f32),
        jax.ShapeDtypeStruct((EPAD, D2), f32),
    ]
    scratch = [
        pltpu.VMEM((QM * CG,), jnp.int32),
        pltpu.VMEM((QM * CG,), jnp.int32),
        pltpu.VMEM((2, CG, D2), f32),
        pltpu.VMEM((2, CG, D2), f32),
        pltpu.SemaphoreType.DMA,
        pltpu.SemaphoreType.DMA,
    ]

    @functools.partial(pl.kernel, mesh=mesh, out_type=out_type,
                       scratch_types=scratch)
    def gather_k(ac_hbm, bc_hbm, row_hbm, col_hbm,
                 ar_hbm, bc_out_hbm,
                 idxr, idxc, buf0, buf1, gsem, osem):
        cid = lax.axis_index("c")
        sid = lax.axis_index("s")
        nch = lax.select(cid == 0, QA, QB)
        tbase = lax.select(cid == 0, sid * (QA * CG),
                           NS * (QA * CG) + sid * (QB * CG))
        tbase = pl.multiple_of(tbase, CG)
        pltpu.sync_copy(row_hbm.at[pl.ds(tbase, QM * CG)], idxr)
        pltpu.sync_copy(col_hbm.at[pl.ds(tbase, QM * CG)], idxc)

        def fire_gather(j, p):
            pltpu.async_copy(ac_hbm.at[idxr.at[pl.ds(j * CG, CG)]],
                             buf0.at[p], gsem)
            pltpu.async_copy(bc_hbm.at[idxc.at[pl.ds(j * CG, CG)]],
                             buf1.at[p], gsem)

        def wait_gather(j, p):
            pltpu.make_async_copy(ac_hbm.at[idxr.at[pl.ds(j * CG, CG)]],
                                  buf0.at[p], gsem).wait()
            pltpu.make_async_copy(bc_hbm.at[idxc.at[pl.ds(j * CG, CG)]],
                                  buf1.at[p], gsem).wait()

        def fire_out(j, p):
            pltpu.async_copy(buf0.at[p], ar_hbm.at[pl.ds(tbase + j * CG, CG)],
                             osem)
            pltpu.async_copy(buf1.at[p],
                             bc_out_hbm.at[pl.ds(tbase + j * CG, CG)], osem)

        def wait_out(j, p):
            pltpu.make_async_copy(buf0.at[p],
                                  ar_hbm.at[pl.ds(tbase + j * CG, CG)],
                                  osem).wait()
            pltpu.make_async_copy(buf1.at[p],
                                  bc_out_hbm.at[pl.ds(tbase + j * CG, CG)],
                                  osem).wait()

        fire_gather(0, 0)

        def body(j, carry):
            cur = lax.rem(j, 2)
            oth = 1 - cur

            @pl.when(j > 0)
            def _():
                wait_out(j - 1, oth)

            @pl.when(j < nch - 1)
            def _():
                fire_gather(j + 1, oth)

            wait_gather(j, cur)
            fire_out(j, cur)
            return carry

        lax.fori_loop(0, nch, body, 0)
        wait_out(nch - 1, 1)

    return gather_k


def _make_sc_scatter(nacc):
    mesh = plsc.VectorSubcoreMesh(core_axis_name="c", subcore_axis_name="s")
    out_type = jax.ShapeDtypeStruct((NC, nacc, D), f32)
    scratch = [
        pltpu.VMEM((NCHS, CS), jnp.int32),
        pltpu.VMEM((2, CS, D), f32),
        pltpu.VMEM_SHARED((nacc, D), f32),
        pltpu.SemaphoreType.DMA,
        pltpu.SemaphoreType.DMA,
    ]
    RPT = nacc // NS

    @functools.partial(pl.kernel, mesh=mesh, out_type=out_type,
                       scratch_types=scratch)
    def scatter_k(ef_hbm, idx2_hbm, zm_hbm, pm_hbm,
                  idxs, bufe, accm, rsem, ssem):
        cid = lax.axis_index("c")
        sid = lax.axis_index("s")
        wid = sid * NC + cid
        pltpu.sync_copy(idx2_hbm.at[pl.ds(wid * NCHS, NCHS)], idxs)
        pltpu.sync_copy(zm_hbm.at[pl.ds(sid * RPT, RPT)],
                        accm.at[pl.ds(sid * RPT, RPT)])
        plsc.subcore_barrier()

        def fire_read(j, p):
            base = wid * EPT + j * CS
            pltpu.async_copy(ef_hbm.at[pl.ds(base, CS)], bufe.at[p], rsem)

        def wait_read(j, p):
            base = wid * EPT + j * CS
            pltpu.make_async_copy(ef_hbm.at[pl.ds(base, CS)], bufe.at[p],
                                  rsem).wait()

        def fire_add(j, p):
            pltpu.async_copy(bufe.at[p], accm.at[idxs.at[j]], ssem, add=True)

        def wait_add(j, p):
            pltpu.make_async_copy(bufe.at[p], accm.at[idxs.at[j]], ssem).wait()

        fire_read(0, 0)

        def body(j, carry):
            cur = lax.rem(j, 2)
            oth = 1 - cur

            @pl.when(j > 0)
            def _():
                wait_add(j - 1, oth)

            @pl.when(j < NCHS - 1)
            def _():
                fire_read(j + 1, oth)

            wait_read(j, cur)
            fire_add(j, cur)
            return carry

        lax.fori_loop(0, NCHS, body, 0)
        wait_add(NCHS - 1, (NCHS - 1) % 2)
        plsc.subcore_barrier()
        pltpu.sync_copy(accm.at[pl.ds(sid * RPT, RPT)],
                        pm_hbm.at[cid, pl.ds(sid * RPT, RPT)])

    return scatter_k


_SC_GATHER = None
_SC_SCATTER_N = None
_SC_SCATTER_8 = None


def _sc_gather(ac, bc, rowg, colg):
    global _SC_GATHER
    if _SC_GATHER is None:
        _SC_GATHER = _make_sc_gather()
    return _SC_GATHER(ac, bc, rowg, colg)


def _sc_scatter_n(ef, row2, zm):
    global _SC_SCATTER_N
    if _SC_SCATTER_N is None:
        _SC_SCATTER_N = _make_sc_scatter(NPAD)
    return _SC_SCATTER_N(ef, row2, zm)


def _sc_scatter_8(sm, row82, zm):
    global _SC_SCATTER_8
    if _SC_SCATTER_8 is None:
        _SC_SCATTER_8 = _make_sc_scatter(N8)
    return _SC_SCATTER_8(sm, row82, zm)


def _tc_linear(x, wT, bias):
    nb = N // BN

    def body(x_r, w_r, b_r, o_r):
        o_r[...] = jnp.dot(x_r[...], w_r[...],
                           preferred_element_type=f32) + b_r[...]

    return pl.pallas_call(
        body,
        grid=(nb,),
        in_specs=[
            pl.BlockSpec((BN, D), lambda p: (p, 0)),
            pl.BlockSpec((D, D), lambda p: (0, 0)),
            pl.BlockSpec((1, D), lambda p: (0, 0)),
        ],
        out_specs=pl.BlockSpec((BN, D), lambda p: (p, 0)),
        out_shape=jax.ShapeDtypeStruct((N, D), f32),
    )(x, wT, bias)


def _tc_pre(h, coord, waT, b1, wbT):
    nb = N // BN

    def body(h_r, c_r, wa_r, b1_r, wb_r, ac_r, bc_r):
        hv = h_r[...]
        cv = c_r[...]
        ac_r[:, :D] = jnp.dot(hv, wa_r[...], preferred_element_type=f32) + b1_r[...]
        ac_r[:, D:] = cv
        bc_r[:, :D] = jnp.dot(hv, wb_r[...], preferred_element_type=f32)
        bc_r[:, D:] = cv

    return pl.pallas_call(
        body,
        grid=(nb,),
        in_specs=[
            pl.BlockSpec((BN, D), lambda p: (p, 0)),
            pl.BlockSpec((BN, D), lambda p: (p, 0)),
            pl.BlockSpec((D, D), lambda p: (0, 0)),
            pl.BlockSpec((1, D), lambda p: (0, 0)),
            pl.BlockSpec((D, D), lambda p: (0, 0)),
        ],
        out_specs=[
            pl.BlockSpec((BN, D2), lambda p: (p, 0)),
            pl.BlockSpec((BN, D2), lambda p: (p, 0)),
        ],
        out_shape=[
            jax.ShapeDtypeStruct((N, D2), f32),
            jax.ShapeDtypeStruct((N, D2), f32),
        ],
    )(h, coord, waT, b1, wbT)


def _tc_edge(acr, bcc, eap, rowe, w1dT, w1c, w2T, b2, w3T, b3, w4):
    nb = EPAD // BE

    def body(ac_r, bc_r, ea_r, row_r,
             w1d_r, w1c_r, w2_r, b2_r, w3_r, b3_r, w4_r,
             ef_o, sm_o):
        p = pl.program_id(0)
        acv = ac_r[...]
        bcv = bc_r[...]
        ar = acv[:, :D]
        cr = acv[:, D:]
        br = bcv[:, :D]
        cc = bcv[:, D:]
        cd = cr - cc
        radial = jnp.sum(cd * cd, axis=1, keepdims=True)
        pre = (ar + br + radial * w1c_r[...]
               + jnp.dot(ea_r[...], w1d_r[...], preferred_element_type=f32))
        m = _silu(pre)
        ef = _silu(jnp.dot(m, w2_r[...], preferred_element_type=f32) + b2_r[...])
        t = _silu(jnp.dot(ef, w3_r[...], preferred_element_type=f32) + b3_r[...])
        w = jnp.sum(t * w4_r[...], axis=1, keepdims=True)
        rowv = row_r[...]
        base_l = 16 * lax.rem(rowv, 8)
        lane = lax.broadcasted_iota(jnp.int32, (BE, D), 1)
        tx = cd[:, 0:1] * w
        ty = cd[:, 1:2] * w
        tz = cd[:, 2:3] * w
        sm = (tx * (lane == base_l) + ty * (lane == base_l + 1)
              + tz * (lane == base_l + 2) + (lane == base_l + 3).astype(f32))
        rowid = p * BE + lax.broadcasted_iota(jnp.int32, (BE, 1), 0)
        maskf = (rowid < E).astype(f32)
        ef_o[...] = ef * maskf
        sm_o[...] = sm * maskf

    return pl.pallas_call(
        body,
        grid=(nb,),
        in_specs=[
            pl.BlockSpec((BE, D2), lambda p: (p, 0)),
            pl.BlockSpec((BE, D2), lambda p: (p, 0)),
            pl.BlockSpec((BE, EA), lambda p: (p, 0)),
            pl.BlockSpec((BE, 1), lambda p: (p, 0)),
            pl.BlockSpec((EA, D), lambda p: (0, 0)),
            pl.BlockSpec((1, D), lambda p: (0, 0)),
            pl.BlockSpec((D, D), lambda p: (0, 0)),
            pl.BlockSpec((1, D), lambda p: (0, 0)),
            pl.BlockSpec((D, D), lambda p: (0, 0)),
            pl.BlockSpec((1, D), lambda p: (0, 0)),
            pl.BlockSpec((1, D), lambda p: (0, 0)),
        ],
        out_specs=[
            pl.BlockSpec((BE, D), lambda p: (p, 0)),
            pl.BlockSpec((BE, D), lambda p: (p, 0)),
        ],
        out_shape=[
            jax.ShapeDtypeStruct((EPAD, D), f32),
            jax.ShapeDtypeStruct((EPAD, D), f32),
        ],
    )(acr, bcc, eap, rowe, w1dT, w1c, w2T, b2, w3T, b3, w4)


def _tc_node(h, coord, pm, ps, wn1aT, wn1bT, bn1, wn2T, bn2):
    BNN = 2048
    BNP8N = BNN // 8
    nb = NPAD // BNN

    def body(h_r, c_r, pm_r, ps_r, wa_r, wb_r, b1_r, w2_r, b2_r,
             ho_r, co_r):
        magg = pm_r[0] + pm_r[1]
        packed = ps_r[0] + ps_r[1]
        ri = lax.broadcasted_iota(jnp.int32, (BNN, BNP8N), 0)
        ci = lax.broadcasted_iota(jnp.int32, (BNN, BNP8N), 1)
        pmat = ((ri // 8) == ci).astype(f32)
        rows_exp = jnp.dot(pmat, packed, preferred_element_type=f32)
        m8 = lax.rem(lax.broadcasted_iota(jnp.int32, (BNN, 1), 0), 8)
        base_l = 16 * m8
        lane = lax.broadcasted_iota(jnp.int32, (BNN, D), 1)
        tx = jnp.sum(jnp.where(lane == base_l, rows_exp, 0.0), axis=1,
                     keepdims=True)
        ty = jnp.sum(jnp.where(lane == base_l + 1, rows_exp, 0.0), axis=1,
                     keepdims=True)
        tz = jnp.sum(jnp.where(lane == base_l + 2, rows_exp, 0.0), axis=1,
                     keepdims=True)
        cnt = jnp.sum(jnp.where(lane == base_l + 3, rows_exp, 0.0), axis=1,
                      keepdims=True)
        agg = (tx * (lane == 0) + ty * (lane == 1) + tz * (lane == 2))
        co_r[...] = c_r[...] + agg / jnp.maximum(cnt, 1.0)
        hv = h_r[...]
        hh = _silu(jnp.dot(hv, wa_r[...], preferred_element_type=f32)
                   + jnp.dot(magg, wb_r[...], preferred_element_type=f32)
                   + b1_r[...])
        ho_r[...] = hv + jnp.dot(hh, w2_r[...], preferred_element_type=f32) + b2_r[...]

    return pl.pallas_call(
        body,
        grid=(nb,),
        in_specs=[
            pl.BlockSpec((BNN, D), lambda p: (p, 0)),
            pl.BlockSpec((BNN, D), lambda p: (p, 0)),
            pl.BlockSpec((NC, BNN, D), lambda p: (0, p, 0)),
            pl.BlockSpec((NC, BNP8N, D), lambda p: (0, p, 0)),
            pl.BlockSpec((D, D), lambda p: (0, 0)),
            pl.BlockSpec((D, D), lambda p: (0, 0)),
            pl.BlockSpec((1, D), lambda p: (0, 0)),
            pl.BlockSpec((D, D), lambda p: (0, 0)),
            pl.BlockSpec((1, D), lambda p: (0, 0)),
        ],
        out_specs=[
            pl.BlockSpec((BNN, D), lambda p: (p, 0)),
            pl.BlockSpec((BNN, D), lambda p: (p, 0)),
        ],
        out_shape=[
            jax.ShapeDtypeStruct((N, D), f32),
            jax.ShapeDtypeStruct((N, D), f32),
        ],
    )(h, coord, pm, ps, wn1aT, wn1bT, bn1, wn2T, bn2)


def _tc_mean(h):
    nb = N // BN

    def body(h_r, o_r):
        p = pl.program_id(0)
        part = jnp.sum(h_r[...], axis=0, keepdims=True) * (1.0 / N)

        @pl.when(p == 0)
        def _():
            o_r[...] = part

        @pl.when(p != 0)
        def _():
            o_r[...] = o_r[...] + part

    return pl.pallas_call(
        body,
        grid=(nb,),
        in_specs=[pl.BlockSpec((BN, D), lambda p: (p, 0))],
        out_specs=pl.BlockSpec((1, D), lambda p: (0, 0)),
        out_shape=jax.ShapeDtypeStruct((1, D), f32),
    )(h)


def kernel(h, x, edges, edge_attr, params):
    row = edges[0].astype(jnp.int32)
    col = edges[1].astype(jnp.int32)
    pad = EPAD - E
    rowp = jnp.concatenate([row, jnp.zeros((pad,), jnp.int32)])
    colp = jnp.concatenate([col, jnp.zeros((pad,), jnp.int32)])
    rowg = jnp.concatenate([rowp, jnp.zeros((IPAD - EPAD,), jnp.int32)])
    colg = jnp.concatenate([colp, jnp.zeros((IPAD - EPAD,), jnp.int32)])
    rowe = rowp.reshape(EPAD, 1)
    row2 = rowp.reshape(EPAD // CS, CS)
    row82 = (rowp // 8).reshape(EPAD // CS, CS)
    eap = jnp.concatenate([edge_attr, jnp.zeros((pad, EA), f32)], axis=0)
    coord = jnp.concatenate([x, jnp.zeros((N, D - 3), f32)], axis=1)
    zm = jnp.zeros((NPAD, D), f32)

    for bp in params:
        wi = bp["emb_in"]
        h = _tc_linear(h, wi["W"].T, wi["b"][None, :])
        for gp in bp["gcls"]:
            w1 = gp["edge_mlp"][0]["W"]
            b1 = gp["edge_mlp"][0]["b"]
            w2 = gp["edge_mlp"][1]["W"]
            b2 = gp["edge_mlp"][1]["b"]
            w3 = gp["coord_mlp"][0]["W"]
            b3 = gp["coord_mlp"][0]["b"]
            w4 = gp["coord_mlp"][1]["W"]
            wn1 = gp["node_mlp"][0]["W"]
            bn1 = gp["node_mlp"][0]["b"]
            wn2 = gp["node_mlp"][1]["W"]
            bn2 = gp["node_mlp"][1]["b"]

            ac, bc = _tc_pre(h, coord, w1[:, :D].T, b1[None, :],
                             w1[:, D:2 * D].T)
            acr, bcc = _sc_gather(ac, bc, rowg, colg)
            ef, sm = _tc_edge(acr, bcc, eap, rowe,
                              w1[:, 2 * D + 1:].T, w1[:, 2 * D][None, :],
                              w2.T, b2[None, :], w3.T, b3[None, :], w4)
            pm = _sc_scatter_n(ef, row2, zm)
            ps = _sc_scatter_8(sm, row82, zm)
            h, coord = _tc_node(h, coord, pm, ps,
                                wn1[:, :D].T, wn1[:, D:].T, bn1[None, :],
                                wn2.T, bn2[None, :])
        wo = bp["emb_out"]
        h = _tc_linear(h, wo["W"].T, wo["b"][None, :])

    mol = _tc_mean(h)
    return (mol, h, coord[:, :3])


# speedup vs baseline: 1.0958x; 1.0749x over previous
import functools

import jax
import jax.numpy as jnp
from jax import lax
from jax.experimental import pallas as pl
from jax.experimental.pallas import tpu as pltpu
from jax.experimental.pallas import tpu_sc as plsc

N = 10000
E = 160000
D = 128
D2 = 256
EA = 16
NC = 2
NS = 16
NW = NC * NS
EPAD = 163840
EHALF = EPAD // 2
EPT = EHALF // NW
CG = 64
NCHG = EPT // CG
CS = 128
NCHS = EPT // CS
NPAD = 10240
N8 = NPAD // 8
BE = 2048
BN = 2000
BNP4 = BN // 4

f32 = jnp.float32


def _silu(v):
    return v * (1.0 / (1.0 + jnp.exp(-v)))


def _make_sc_gather():
    mesh = plsc.VectorSubcoreMesh(core_axis_name="c", subcore_axis_name="s")
    out_type = [
        jax.ShapeDtypeStruct((EHALF, D2), f32),
        jax.ShapeDtypeStruct((EHALF, D2), f32),
    ]
    scratch = [
        pltpu.VMEM((EPT,), jnp.int32),
        pltpu.VMEM((EPT,), jnp.int32),
        pltpu.VMEM((2, CG, D2), f32),
        pltpu.VMEM((2, CG, D2), f32),
        pltpu.SemaphoreType.DMA,
        pltpu.SemaphoreType.DMA,
    ]

    @functools.partial(pl.kernel, mesh=mesh, out_type=out_type,
                       scratch_types=scratch)
    def gather_k(ac_hbm, bc_hbm, row_hbm, col_hbm,
                 ar_hbm, bc_out_hbm,
                 idxr, idxc, buf0, buf1, gsem, osem):
        cid = lax.axis_index("c")
        sid = lax.axis_index("s")
        nch = NCHG
        wid = sid * NC + cid
        tbase = wid * EPT
        pltpu.sync_copy(row_hbm.at[pl.ds(tbase, EPT)], idxr)
        pltpu.sync_copy(col_hbm.at[pl.ds(tbase, EPT)], idxc)

        def fire_gather(j, p):
            pltpu.async_copy(ac_hbm.at[idxr.at[pl.ds(j * CG, CG)]],
                             buf0.at[p], gsem)
            pltpu.async_copy(bc_hbm.at[idxc.at[pl.ds(j * CG, CG)]],
                             buf1.at[p], gsem)

        def wait_gather(j, p):
            pltpu.make_async_copy(ac_hbm.at[idxr.at[pl.ds(j * CG, CG)]],
                                  buf0.at[p], gsem).wait()
            pltpu.make_async_copy(bc_hbm.at[idxc.at[pl.ds(j * CG, CG)]],
                                  buf1.at[p], gsem).wait()

        def fire_out(j, p):
            pltpu.async_copy(buf0.at[p], ar_hbm.at[pl.ds(tbase + j * CG, CG)],
                             osem)
            pltpu.async_copy(buf1.at[p],
                             bc_out_hbm.at[pl.ds(tbase + j * CG, CG)], osem)

        def wait_out(j, p):
            pltpu.make_async_copy(buf0.at[p],
                                  ar_hbm.at[pl.ds(tbase + j * CG, CG)],
                                  osem).wait()
            pltpu.make_async_copy(buf1.at[p],
                                  bc_out_hbm.at[pl.ds(tbase + j * CG, CG)],
                                  osem).wait()

        fire_gather(0, 0)

        def body(j, carry):
            cur = lax.rem(j, 2)
            oth = 1 - cur

            @pl.when(j > 0)
            def _():
                wait_out(j - 1, oth)

            @pl.when(j < nch - 1)
            def _():
                fire_gather(j + 1, oth)

            wait_gather(j, cur)
            fire_out(j, cur)
            return carry

        lax.fori_loop(0, nch, body, 0)
        wait_out(nch - 1, 1)

    return gather_k


def _make_sc_scatter(nacc):
    mesh = plsc.VectorSubcoreMesh(core_axis_name="c", subcore_axis_name="s")
    out_type = jax.ShapeDtypeStruct((NC, nacc, D), f32)
    scratch = [
        pltpu.VMEM((NCHS, CS), jnp.int32),
        pltpu.VMEM((2, CS, D), f32),
        pltpu.VMEM_SHARED((nacc, D), f32),
        pltpu.SemaphoreType.DMA,
        pltpu.SemaphoreType.DMA,
    ]
    RPT = nacc // NS

    @functools.partial(pl.kernel, mesh=mesh, out_type=out_type,
                       scratch_types=scratch)
    def scatter_k(ef_hbm, idx2_hbm, zm_hbm, pm_hbm,
                  idxs, bufe, accm, rsem, ssem):
        cid = lax.axis_index("c")
        sid = lax.axis_index("s")
        wid = sid * NC + cid
        pltpu.sync_copy(idx2_hbm.at[wid], idxs)
        pltpu.sync_copy(zm_hbm.at[pl.ds(sid * RPT, RPT)],
                        accm.at[pl.ds(sid * RPT, RPT)])
        plsc.subcore_barrier()

        def fire_read(j, p):
            base = wid * EPT + j * CS
            pltpu.async_copy(ef_hbm.at[pl.ds(base, CS)], bufe.at[p], rsem)

        def wait_read(j, p):
            base = wid * EPT + j * CS
            pltpu.make_async_copy(ef_hbm.at[pl.ds(base, CS)], bufe.at[p],
                                  rsem).wait()

        def fire_add(j, p):
            pltpu.async_copy(bufe.at[p], accm.at[idxs.at[j]], ssem, add=True)

        def wait_add(j, p):
            pltpu.make_async_copy(bufe.at[p], accm.at[idxs.at[j]], ssem).wait()

        fire_read(0, 0)

        def body(j, carry):
            cur = lax.rem(j, 2)
            oth = 1 - cur

            @pl.when(j > 0)
            def _():
                wait_add(j - 1, oth)

            @pl.when(j < NCHS - 1)
            def _():
                fire_read(j + 1, oth)

            wait_read(j, cur)
            fire_add(j, cur)
            return carry

        lax.fori_loop(0, NCHS, body, 0)
        wait_add(NCHS - 1, (NCHS - 1) % 2)
        plsc.subcore_barrier()
        pltpu.sync_copy(accm.at[pl.ds(sid * RPT, RPT)],
                        pm_hbm.at[cid, pl.ds(sid * RPT, RPT)])

    return scatter_k


_SC_GATHER = None
_SC_SCATTER_N = None
_SC_SCATTER_8 = None


def _sc_gather(ac, bc, rowh, colh):
    global _SC_GATHER
    if _SC_GATHER is None:
        _SC_GATHER = _make_sc_gather()
    return _SC_GATHER(ac, bc, rowh, colh)


def _sc_scatter_n(ef, row2, zm):
    global _SC_SCATTER_N
    if _SC_SCATTER_N is None:
        _SC_SCATTER_N = _make_sc_scatter(NPAD)
    return _SC_SCATTER_N(ef, row2, zm)


def _sc_scatter_8(sm, row82, zm):
    global _SC_SCATTER_8
    if _SC_SCATTER_8 is None:
        _SC_SCATTER_8 = _make_sc_scatter(N8)
    return _SC_SCATTER_8(sm, row82, zm)


def _tc_linear(x, wT, bias):
    nb = N // BN

    def body(x_r, w_r, b_r, o_r):
        o_r[...] = jnp.dot(x_r[...], w_r[...],
                           preferred_element_type=f32) + b_r[...]

    return pl.pallas_call(
        body,
        grid=(nb,),
        in_specs=[
            pl.BlockSpec((BN, D), lambda p: (p, 0)),
            pl.BlockSpec((D, D), lambda p: (0, 0)),
            pl.BlockSpec((1, D), lambda p: (0, 0)),
        ],
        out_specs=pl.BlockSpec((BN, D), lambda p: (p, 0)),
        out_shape=jax.ShapeDtypeStruct((N, D), f32),
    )(x, wT, bias)


def _tc_pre(h, coord, waT, b1, wbT):
    nb = N // BN

    def body(h_r, c_r, wa_r, b1_r, wb_r, ac_r, bc_r):
        hv = h_r[...]
        cv = c_r[...]
        ac_r[:, :D] = jnp.dot(hv, wa_r[...], preferred_element_type=f32) + b1_r[...]
        ac_r[:, D:] = cv
        bc_r[:, :D] = jnp.dot(hv, wb_r[...], preferred_element_type=f32)
        bc_r[:, D:] = cv

    return pl.pallas_call(
        body,
        grid=(nb,),
        in_specs=[
            pl.BlockSpec((BN, D), lambda p: (p, 0)),
            pl.BlockSpec((BN, D), lambda p: (p, 0)),
            pl.BlockSpec((D, D), lambda p: (0, 0)),
            pl.BlockSpec((1, D), lambda p: (0, 0)),
            pl.BlockSpec((D, D), lambda p: (0, 0)),
        ],
        out_specs=[
            pl.BlockSpec((BN, D2), lambda p: (p, 0)),
            pl.BlockSpec((BN, D2), lambda p: (p, 0)),
        ],
        out_shape=[
            jax.ShapeDtypeStruct((N, D2), f32),
            jax.ShapeDtypeStruct((N, D2), f32),
        ],
    )(h, coord, waT, b1, wbT)


def _tc_edge(acr, bcc, eap, rowe, w1dT, w1c, w2T, b2, w3T, b3, w4, elim):
    nb = EHALF // BE

    def body(ac_r, bc_r, ea_r, row_r,
             w1d_r, w1c_r, w2_r, b2_r, w3_r, b3_r, w4_r,
             ef_o, sm_o):
        p = pl.program_id(0)
        acv = ac_r[...]
        bcv = bc_r[...]
        ar = acv[:, :D]
        cr = acv[:, D:]
        br = bcv[:, :D]
        cc = bcv[:, D:]
        cd = cr - cc
        radial = jnp.sum(cd * cd, axis=1, keepdims=True)
        pre = (ar + br + radial * w1c_r[...]
               + jnp.dot(ea_r[...], w1d_r[...], preferred_element_type=f32))
        m = _silu(pre)
        ef = _silu(jnp.dot(m, w2_r[...], preferred_element_type=f32) + b2_r[...])
        t = _silu(jnp.dot(ef, w3_r[...], preferred_element_type=f32) + b3_r[...])
        w = jnp.sum(t * w4_r[...], axis=1, keepdims=True)
        rowv = row_r[...]
        base_l = 16 * lax.rem(rowv, 8)
        lane = lax.broadcasted_iota(jnp.int32, (BE, D), 1)
        tx = cd[:, 0:1] * w
        ty = cd[:, 1:2] * w
        tz = cd[:, 2:3] * w
        sm = (tx * (lane == base_l) + ty * (lane == base_l + 1)
              + tz * (lane == base_l + 2) + (lane == base_l + 3).astype(f32))
        rowid = p * BE + lax.broadcasted_iota(jnp.int32, (BE, 1), 0)
        maskf = (rowid < elim).astype(f32)
        ef_o[...] = ef * maskf
        sm_o[...] = sm * maskf

    return pl.pallas_call(
        body,
        grid=(nb,),
        in_specs=[
            pl.BlockSpec((BE, D2), lambda p: (p, 0)),
            pl.BlockSpec((BE, D2), lambda p: (p, 0)),
            pl.BlockSpec((BE, EA), lambda p: (p, 0)),
            pl.BlockSpec((BE, 1), lambda p: (p, 0)),
            pl.BlockSpec((EA, D), lambda p: (0, 0)),
            pl.BlockSpec((1, D), lambda p: (0, 0)),
            pl.BlockSpec((D, D), lambda p: (0, 0)),
            pl.BlockSpec((1, D), lambda p: (0, 0)),
            pl.BlockSpec((D, D), lambda p: (0, 0)),
            pl.BlockSpec((1, D), lambda p: (0, 0)),
            pl.BlockSpec((1, D), lambda p: (0, 0)),
        ],
        out_specs=[
            pl.BlockSpec((BE, D), lambda p: (p, 0)),
            pl.BlockSpec((BE, D), lambda p: (p, 0)),
        ],
        out_shape=[
            jax.ShapeDtypeStruct((EHALF, D), f32),
            jax.ShapeDtypeStruct((EHALF, D), f32),
        ],
    )(acr, bcc, eap, rowe, w1dT, w1c, w2T, b2, w3T, b3, w4)


def _tc_node(h, coord, pm0, pm1, ps0, ps1, wn1aT, wn1bT, bn1, wn2T, bn2):
    BNN = 2048
    BNP8N = BNN // 8
    nb = NPAD // BNN

    def body(h_r, c_r, pm0_r, pm1_r, ps0_r, ps1_r,
             wa_r, wb_r, b1_r, w2_r, b2_r,
             ho_r, co_r):
        magg = pm0_r[0] + pm0_r[1] + pm1_r[0] + pm1_r[1]
        packed = ps0_r[0] + ps0_r[1] + ps1_r[0] + ps1_r[1]
        ri = lax.broadcasted_iota(jnp.int32, (BNN, BNP8N), 0)
        ci = lax.broadcasted_iota(jnp.int32, (BNN, BNP8N), 1)
        pmat = ((ri // 8) == ci).astype(f32)
        rows_exp = jnp.dot(pmat, packed, preferred_element_type=f32)
        m8 = lax.rem(lax.broadcasted_iota(jnp.int32, (BNN, 1), 0), 8)
        base_l = 16 * m8
        lane = lax.broadcasted_iota(jnp.int32, (BNN, D), 1)
        tx = jnp.sum(jnp.where(lane == base_l, rows_exp, 0.0), axis=1,
                     keepdims=True)
        ty = jnp.sum(jnp.where(lane == base_l + 1, rows_exp, 0.0), axis=1,
                     keepdims=True)
        tz = jnp.sum(jnp.where(lane == base_l + 2, rows_exp, 0.0), axis=1,
                     keepdims=True)
        cnt = jnp.sum(jnp.where(lane == base_l + 3, rows_exp, 0.0), axis=1,
                      keepdims=True)
        agg = (tx * (lane == 0) + ty * (lane == 1) + tz * (lane == 2))
        co_r[...] = c_r[...] + agg / jnp.maximum(cnt, 1.0)
        hv = h_r[...]
        hh = _silu(jnp.dot(hv, wa_r[...], preferred_element_type=f32)
                   + jnp.dot(magg, wb_r[...], preferred_element_type=f32)
                   + b1_r[...])
        ho_r[...] = hv + jnp.dot(hh, w2_r[...], preferred_element_type=f32) + b2_r[...]

    return pl.pallas_call(
        body,
        grid=(nb,),
        in_specs=[
            pl.BlockSpec((BNN, D), lambda p: (p, 0)),
            pl.BlockSpec((BNN, D), lambda p: (p, 0)),
            pl.BlockSpec((NC, BNN, D), lambda p: (0, p, 0)),
            pl.BlockSpec((NC, BNN, D), lambda p: (0, p, 0)),
            pl.BlockSpec((NC, BNP8N, D), lambda p: (0, p, 0)),
            pl.BlockSpec((NC, BNP8N, D), lambda p: (0, p, 0)),
            pl.BlockSpec((D, D), lambda p: (0, 0)),
            pl.BlockSpec((D, D), lambda p: (0, 0)),
            pl.BlockSpec((1, D), lambda p: (0, 0)),
            pl.BlockSpec((D, D), lambda p: (0, 0)),
            pl.BlockSpec((1, D), lambda p: (0, 0)),
        ],
        out_specs=[
            pl.BlockSpec((BNN, D), lambda p: (p, 0)),
            pl.BlockSpec((BNN, D), lambda p: (p, 0)),
        ],
        out_shape=[
            jax.ShapeDtypeStruct((N, D), f32),
            jax.ShapeDtypeStruct((N, D), f32),
        ],
    )(h, coord, pm0, pm1, ps0, ps1, wn1aT, wn1bT, bn1, wn2T, bn2)


def _tc_mean(h):
    nb = N // BN

    def body(h_r, o_r):
        p = pl.program_id(0)
        part = jnp.sum(h_r[...], axis=0, keepdims=True) * (1.0 / N)

        @pl.when(p == 0)
        def _():
            o_r[...] = part

        @pl.when(p != 0)
        def _():
            o_r[...] = o_r[...] + part

    return pl.pallas_call(
        body,
        grid=(nb,),
        in_specs=[pl.BlockSpec((BN, D), lambda p: (p, 0))],
        out_specs=pl.BlockSpec((1, D), lambda p: (0, 0)),
        out_shape=jax.ShapeDtypeStruct((1, D), f32),
    )(h)


def kernel(h, x, edges, edge_attr, params):
    row = edges[0].astype(jnp.int32)
    col = edges[1].astype(jnp.int32)
    pad = EPAD - E
    rowp = jnp.concatenate([row, jnp.zeros((pad,), jnp.int32)])
    colp = jnp.concatenate([col, jnp.zeros((pad,), jnp.int32)])
    rowh = (rowp[:EHALF], rowp[EHALF:])
    colh = (colp[:EHALF], colp[EHALF:])
    rowe = tuple(r.reshape(EHALF, 1) for r in rowh)
    row2 = tuple(r.reshape(NW, NCHS, CS) for r in rowh)
    row82 = tuple((r // 8).reshape(NW, NCHS, CS) for r in rowh)
    eapf = jnp.concatenate([edge_attr, jnp.zeros((pad, EA), f32)], axis=0)
    eap = (eapf[:EHALF], eapf[EHALF:])
    elim = (min(E, EHALF), max(0, E - EHALF))
    coord = jnp.concatenate([x, jnp.zeros((N, D - 3), f32)], axis=1)
    zm = jnp.zeros((NPAD, D), f32)

    for bp in params:
        wi = bp["emb_in"]
        h = _tc_linear(h, wi["W"].T, wi["b"][None, :])
        for gp in bp["gcls"]:
            w1 = gp["edge_mlp"][0]["W"]
            b1 = gp["edge_mlp"][0]["b"]
            w2 = gp["edge_mlp"][1]["W"]
            b2 = gp["edge_mlp"][1]["b"]
            w3 = gp["coord_mlp"][0]["W"]
            b3 = gp["coord_mlp"][0]["b"]
            w4 = gp["coord_mlp"][1]["W"]
            wn1 = gp["node_mlp"][0]["W"]
            bn1 = gp["node_mlp"][0]["b"]
            wn2 = gp["node_mlp"][1]["W"]
            bn2 = gp["node_mlp"][1]["b"]

            ac, bc = _tc_pre(h, coord, w1[:, :D].T, b1[None, :],
                             w1[:, D:2 * D].T)
            pm, ps = [], []
            for hf in (0, 1):
                acr, bcc = _sc_gather(ac, bc, rowh[hf], colh[hf])
                ef, sm = _tc_edge(acr, bcc, eap[hf], rowe[hf],
                                  w1[:, 2 * D + 1:].T, w1[:, 2 * D][None, :],
                                  w2.T, b2[None, :], w3.T, b3[None, :], w4,
                                  elim[hf])
                pm.append(_sc_scatter_n(ef, row2[hf], zm))
                ps.append(_sc_scatter_8(sm, row82[hf], zm))
            h, coord = _tc_node(h, coord, pm[0], pm[1], ps[0], ps[1],
                                wn1[:, :D].T, wn1[:, D:].T, bn1[None, :],
                                wn2.T, bn2[None, :])
        wo = bp["emb_out"]
        h = _tc_linear(h, wo["W"].T, wo["b"][None, :])

    mol = _tc_mean(h)
    return (mol, h, coord[:, :3])


# speedup vs baseline: 1.1063x; 1.0096x over previous
import functools

import jax
import jax.numpy as jnp
from jax import lax
from jax.experimental import pallas as pl
from jax.experimental.pallas import tpu as pltpu
from jax.experimental.pallas import tpu_sc as plsc

N = 10000
E = 160000
D = 128
D2 = 256
EA = 16
NC = 2
NS = 16
NW = NC * NS
EPAD = 163840
EHALF = EPAD // 2
EPT = EHALF // NW
CS = 128
EPT2 = EHALF // 16
NCH2 = EPT2 // CS
NCHS = EPT // CS
NPAD = 10240
N8 = NPAD // 8
BE = 2048
BN = 2000
BNP4 = BN // 4

f32 = jnp.float32


def _silu(v):
    return v * (1.0 / (1.0 + jnp.exp(-v)))


def _make_sc_gather():
    mesh = plsc.VectorSubcoreMesh(core_axis_name="c", subcore_axis_name="s")
    out_type = [
        jax.ShapeDtypeStruct((EHALF, D2), f32),
        jax.ShapeDtypeStruct((EHALF, D2), f32),
    ]
    scratch = [
        pltpu.VMEM((EPT2,), jnp.int32),
        pltpu.VMEM((2, CS, D2), f32),
        pltpu.SemaphoreType.DMA,
        pltpu.SemaphoreType.DMA,
    ]

    @functools.partial(pl.kernel, mesh=mesh, out_type=out_type,
                       scratch_types=scratch)
    def gather_k(ac_hbm, bc_hbm, row_hbm, col_hbm,
                 ar_hbm, bc_out_hbm,
                 idx, buf, gsem, osem):
        cid = lax.axis_index("c")
        sid = lax.axis_index("s")
        wid = sid * NC + cid
        tbase = pl.multiple_of(lax.rem(wid, 16) * EPT2, CS)

        def run(tbl_hbm, idx_hbm, out_hbm):
            pltpu.sync_copy(idx_hbm.at[pl.ds(tbase, EPT2)], idx)

            def fire_g(j, p):
                pltpu.async_copy(tbl_hbm.at[idx.at[pl.ds(j * CS, CS)]],
                                 buf.at[p], gsem)

            def wait_g(j, p):
                pltpu.make_async_copy(tbl_hbm.at[idx.at[pl.ds(j * CS, CS)]],
                                      buf.at[p], gsem).wait()

            def fire_o(j, p):
                pltpu.async_copy(buf.at[p],
                                 out_hbm.at[pl.ds(tbase + j * CS, CS)], osem)

            def wait_o(j, p):
                pltpu.make_async_copy(buf.at[p],
                                      out_hbm.at[pl.ds(tbase + j * CS, CS)],
                                      osem).wait()

            fire_g(0, 0)

            def body(j, carry):
                cur = lax.rem(j, 2)
                oth = 1 - cur

                @pl.when(j > 0)
                def _():
                    wait_o(j - 1, oth)

                @pl.when(j < NCH2 - 1)
                def _():
                    fire_g(j + 1, oth)

                wait_g(j, cur)
                fire_o(j, cur)
                return carry

            lax.fori_loop(0, NCH2, body, 0)
            wait_o(NCH2 - 1, (NCH2 - 1) % 2)

        @pl.when(wid < 16)
        def _():
            run(ac_hbm, row_hbm, ar_hbm)

        @pl.when(wid >= 16)
        def _():
            run(bc_hbm, col_hbm, bc_out_hbm)

    return gather_k


def _make_sc_scatter(nacc):
    mesh = plsc.VectorSubcoreMesh(core_axis_name="c", subcore_axis_name="s")
    out_type = jax.ShapeDtypeStruct((NC, nacc, D), f32)
    scratch = [
        pltpu.VMEM((NCHS, CS), jnp.int32),
        pltpu.VMEM((2, CS, D), f32),
        pltpu.VMEM_SHARED((nacc, D), f32),
        pltpu.SemaphoreType.DMA,
        pltpu.SemaphoreType.DMA,
    ]
    RPT = nacc // NS

    @functools.partial(pl.kernel, mesh=mesh, out_type=out_type,
                       scratch_types=scratch)
    def scatter_k(ef_hbm, idx2_hbm, zm_hbm, pm_hbm,
                  idxs, bufe, accm, rsem, ssem):
        cid = lax.axis_index("c")
        sid = lax.axis_index("s")
        wid = sid * NC + cid
        pltpu.sync_copy(idx2_hbm.at[wid], idxs)
        pltpu.sync_copy(zm_hbm.at[pl.ds(sid * RPT, RPT)],
                        accm.at[pl.ds(sid * RPT, RPT)])
        plsc.subcore_barrier()

        def fire_read(j, p):
            base = wid * EPT + j * CS
            pltpu.async_copy(ef_hbm.at[pl.ds(base, CS)], bufe.at[p], rsem)

        def wait_read(j, p):
            base = wid * EPT + j * CS
            pltpu.make_async_copy(ef_hbm.at[pl.ds(base, CS)], bufe.at[p],
                                  rsem).wait()

        def fire_add(j, p):
            pltpu.async_copy(bufe.at[p], accm.at[idxs.at[j]], ssem, add=True)

        def wait_add(j, p):
            pltpu.make_async_copy(bufe.at[p], accm.at[idxs.at[j]], ssem).wait()

        fire_read(0, 0)

        def body(j, carry):
            cur = lax.rem(j, 2)
            oth = 1 - cur

            @pl.when(j > 0)
            def _():
                wait_add(j - 1, oth)

            @pl.when(j < NCHS - 1)
            def _():
                fire_read(j + 1, oth)

            wait_read(j, cur)
            fire_add(j, cur)
            return carry

        lax.fori_loop(0, NCHS, body, 0)
        wait_add(NCHS - 1, (NCHS - 1) % 2)
        plsc.subcore_barrier()
        pltpu.sync_copy(accm.at[pl.ds(sid * RPT, RPT)],
                        pm_hbm.at[cid, pl.ds(sid * RPT, RPT)])

    return scatter_k


_SC_GATHER = None
_SC_SCATTER_N = None
_SC_SCATTER_8 = None


def _sc_gather(ac, bc, rowh, colh):
    global _SC_GATHER
    if _SC_GATHER is None:
        _SC_GATHER = _make_sc_gather()
    return _SC_GATHER(ac, bc, rowh, colh)


def _sc_scatter_n(ef, row2, zm):
    global _SC_SCATTER_N
    if _SC_SCATTER_N is None:
        _SC_SCATTER_N = _make_sc_scatter(NPAD)
    return _SC_SCATTER_N(ef, row2, zm)


def _sc_scatter_8(sm, row82, zm):
    global _SC_SCATTER_8
    if _SC_SCATTER_8 is None:
        _SC_SCATTER_8 = _make_sc_scatter(N8)
    return _SC_SCATTER_8(sm, row82, zm)


def _tc_linear(x, wT, bias):
    nb = N // BN

    def body(x_r, w_r, b_r, o_r):
        o_r[...] = jnp.dot(x_r[...], w_r[...],
                           preferred_element_type=f32) + b_r[...]

    return pl.pallas_call(
        body,
        grid=(nb,),
        in_specs=[
            pl.BlockSpec((BN, D), lambda p: (p, 0)),
            pl.BlockSpec((D, D), lambda p: (0, 0)),
            pl.BlockSpec((1, D), lambda p: (0, 0)),
        ],
        out_specs=pl.BlockSpec((BN, D), lambda p: (p, 0)),
        out_shape=jax.ShapeDtypeStruct((N, D), f32),
    )(x, wT, bias)


def _tc_pre(h, coord, waT, b1, wbT):
    nb = N // BN

    def body(h_r, c_r, wa_r, b1_r, wb_r, ac_r, bc_r):
        hv = h_r[...]
        cv = c_r[...]
        ac_r[:, :D] = jnp.dot(hv, wa_r[...], preferred_element_type=f32) + b1_r[...]
        ac_r[:, D:] = cv
        bc_r[:, :D] = jnp.dot(hv, wb_r[...], preferred_element_type=f32)
        bc_r[:, D:] = cv

    return pl.pallas_call(
        body,
        grid=(nb,),
        in_specs=[
            pl.BlockSpec((BN, D), lambda p: (p, 0)),
            pl.BlockSpec((BN, D), lambda p: (p, 0)),
            pl.BlockSpec((D, D), lambda p: (0, 0)),
            pl.BlockSpec((1, D), lambda p: (0, 0)),
            pl.BlockSpec((D, D), lambda p: (0, 0)),
        ],
        out_specs=[
            pl.BlockSpec((BN, D2), lambda p: (p, 0)),
            pl.BlockSpec((BN, D2), lambda p: (p, 0)),
        ],
        out_shape=[
            jax.ShapeDtypeStruct((N, D2), f32),
            jax.ShapeDtypeStruct((N, D2), f32),
        ],
    )(h, coord, waT, b1, wbT)


def _tc_edge(acr, bcc, eap, rowe, w1dT, w1c, w2T, b2, w3T, b3, w4, elim):
    nb = EHALF // BE

    def body(ac_r, bc_r, ea_r, row_r,
             w1d_r, w1c_r, w2_r, b2_r, w3_r, b3_r, w4_r,
             ef_o, sm_o):
        p = pl.program_id(0)
        acv = ac_r[...]
        bcv = bc_r[...]
        ar = acv[:, :D]
        cr = acv[:, D:]
        br = bcv[:, :D]
        cc = bcv[:, D:]
        cd = cr - cc
        radial = jnp.sum(cd * cd, axis=1, keepdims=True)
        pre = (ar + br + radial * w1c_r[...]
               + jnp.dot(ea_r[...], w1d_r[...], preferred_element_type=f32))
        m = _silu(pre)
        ef = _silu(jnp.dot(m, w2_r[...], preferred_element_type=f32) + b2_r[...])
        t = _silu(jnp.dot(ef, w3_r[...], preferred_element_type=f32) + b3_r[...])
        w = jnp.sum(t * w4_r[...], axis=1, keepdims=True)
        rowv = row_r[...]
        base_l = 16 * lax.rem(rowv, 8)
        lane = lax.broadcasted_iota(jnp.int32, (BE, D), 1)
        tx = cd[:, 0:1] * w
        ty = cd[:, 1:2] * w
        tz = cd[:, 2:3] * w
        sm = (tx * (lane == base_l) + ty * (lane == base_l + 1)
              + tz * (lane == base_l + 2) + (lane == base_l + 3).astype(f32))
        rowid = p * BE + lax.broadcasted_iota(jnp.int32, (BE, 1), 0)
        maskf = (rowid < elim).astype(f32)
        ef_o[...] = ef * maskf
        sm_o[...] = sm * maskf

    return pl.pallas_call(
        body,
        grid=(nb,),
        in_specs=[
            pl.BlockSpec((BE, D2), lambda p: (p, 0)),
            pl.BlockSpec((BE, D2), lambda p: (p, 0)),
            pl.BlockSpec((BE, EA), lambda p: (p, 0)),
            pl.BlockSpec((BE, 1), lambda p: (p, 0)),
            pl.BlockSpec((EA, D), lambda p: (0, 0)),
            pl.BlockSpec((1, D), lambda p: (0, 0)),
            pl.BlockSpec((D, D), lambda p: (0, 0)),
            pl.BlockSpec((1, D), lambda p: (0, 0)),
            pl.BlockSpec((D, D), lambda p: (0, 0)),
            pl.BlockSpec((1, D), lambda p: (0, 0)),
            pl.BlockSpec((1, D), lambda p: (0, 0)),
        ],
        out_specs=[
            pl.BlockSpec((BE, D), lambda p: (p, 0)),
            pl.BlockSpec((BE, D), lambda p: (p, 0)),
        ],
        out_shape=[
            jax.ShapeDtypeStruct((EHALF, D), f32),
            jax.ShapeDtypeStruct((EHALF, D), f32),
        ],
    )(acr, bcc, eap, rowe, w1dT, w1c, w2T, b2, w3T, b3, w4)


def _tc_node(h, coord, pm0, pm1, ps0, ps1, wn1aT, wn1bT, bn1, wn2T, bn2):
    BNN = 2048
    BNP8N = BNN // 8
    nb = NPAD // BNN

    def body(h_r, c_r, pm0_r, pm1_r, ps0_r, ps1_r,
             wa_r, wb_r, b1_r, w2_r, b2_r,
             ho_r, co_r):
        magg = pm0_r[0] + pm0_r[1] + pm1_r[0] + pm1_r[1]
        packed = ps0_r[0] + ps0_r[1] + ps1_r[0] + ps1_r[1]
        ri = lax.broadcasted_iota(jnp.int32, (BNN, BNP8N), 0)
        ci = lax.broadcasted_iota(jnp.int32, (BNN, BNP8N), 1)
        pmat = ((ri // 8) == ci).astype(f32)
        rows_exp = jnp.dot(pmat, packed, preferred_element_type=f32)
        m8 = lax.rem(lax.broadcasted_iota(jnp.int32, (BNN, 1), 0), 8)
        base_l = 16 * m8
        lane = lax.broadcasted_iota(jnp.int32, (BNN, D), 1)
        tx = jnp.sum(jnp.where(lane == base_l, rows_exp, 0.0), axis=1,
                     keepdims=True)
        ty = jnp.sum(jnp.where(lane == base_l + 1, rows_exp, 0.0), axis=1,
                     keepdims=True)
        tz = jnp.sum(jnp.where(lane == base_l + 2, rows_exp, 0.0), axis=1,
                     keepdims=True)
        cnt = jnp.sum(jnp.where(lane == base_l + 3, rows_exp, 0.0), axis=1,
                      keepdims=True)
        agg = (tx * (lane == 0) + ty * (lane == 1) + tz * (lane == 2))
        co_r[...] = c_r[...] + agg / jnp.maximum(cnt, 1.0)
        hv = h_r[...]
        hh = _silu(jnp.dot(hv, wa_r[...], preferred_element_type=f32)
                   + jnp.dot(magg, wb_r[...], preferred_element_type=f32)
                   + b1_r[...])
        ho_r[...] = hv + jnp.dot(hh, w2_r[...], preferred_element_type=f32) + b2_r[...]

    return pl.pallas_call(
        body,
        grid=(nb,),
        in_specs=[
            pl.BlockSpec((BNN, D), lambda p: (p, 0)),
            pl.BlockSpec((BNN, D), lambda p: (p, 0)),
            pl.BlockSpec((NC, BNN, D), lambda p: (0, p, 0)),
            pl.BlockSpec((NC, BNN, D), lambda p: (0, p, 0)),
            pl.BlockSpec((NC, BNP8N, D), lambda p: (0, p, 0)),
            pl.BlockSpec((NC, BNP8N, D), lambda p: (0, p, 0)),
            pl.BlockSpec((D, D), lambda p: (0, 0)),
            pl.BlockSpec((D, D), lambda p: (0, 0)),
            pl.BlockSpec((1, D), lambda p: (0, 0)),
            pl.BlockSpec((D, D), lambda p: (0, 0)),
            pl.BlockSpec((1, D), lambda p: (0, 0)),
        ],
        out_specs=[
            pl.BlockSpec((BNN, D), lambda p: (p, 0)),
            pl.BlockSpec((BNN, D), lambda p: (p, 0)),
        ],
        out_shape=[
            jax.ShapeDtypeStruct((N, D), f32),
            jax.ShapeDtypeStruct((N, D), f32),
        ],
    )(h, coord, pm0, pm1, ps0, ps1, wn1aT, wn1bT, bn1, wn2T, bn2)


def _tc_mean(h):
    nb = N // BN

    def body(h_r, o_r):
        p = pl.program_id(0)
        part = jnp.sum(h_r[...], axis=0, keepdims=True) * (1.0 / N)

        @pl.when(p == 0)
        def _():
            o_r[...] = part

        @pl.when(p != 0)
        def _():
            o_r[...] = o_r[...] + part

    return pl.pallas_call(
        body,
        grid=(nb,),
        in_specs=[pl.BlockSpec((BN, D), lambda p: (p, 0))],
        out_specs=pl.BlockSpec((1, D), lambda p: (0, 0)),
        out_shape=jax.ShapeDtypeStruct((1, D), f32),
    )(h)


def kernel(h, x, edges, edge_attr, params):
    row = edges[0].astype(jnp.int32)
    col = edges[1].astype(jnp.int32)
    pad = EPAD - E
    rowp = jnp.concatenate([row, jnp.zeros((pad,), jnp.int32)])
    colp = jnp.concatenate([col, jnp.zeros((pad,), jnp.int32)])
    rowh = (rowp[:EHALF], rowp[EHALF:])
    colh = (colp[:EHALF], colp[EHALF:])
    rowe = tuple(r.reshape(EHALF, 1) for r in rowh)
    row2 = tuple(r.reshape(NW, NCHS, CS) for r in rowh)
    row82 = tuple((r // 8).reshape(NW, NCHS, CS) for r in rowh)
    eapf = jnp.concatenate([edge_attr, jnp.zeros((pad, EA), f32)], axis=0)
    eap = (eapf[:EHALF], eapf[EHALF:])
    elim = (min(E, EHALF), max(0, E - EHALF))
    coord = jnp.concatenate([x, jnp.zeros((N, D - 3), f32)], axis=1)
    zm = jnp.zeros((NPAD, D), f32)

    for bp in params:
        wi = bp["emb_in"]
        h = _tc_linear(h, wi["W"].T, wi["b"][None, :])
        for gp in bp["gcls"]:
            w1 = gp["edge_mlp"][0]["W"]
            b1 = gp["edge_mlp"][0]["b"]
            w2 = gp["edge_mlp"][1]["W"]
            b2 = gp["edge_mlp"][1]["b"]
            w3 = gp["coord_mlp"][0]["W"]
            b3 = gp["coord_mlp"][0]["b"]
            w4 = gp["coord_mlp"][1]["W"]
            wn1 = gp["node_mlp"][0]["W"]
            bn1 = gp["node_mlp"][0]["b"]
            wn2 = gp["node_mlp"][1]["W"]
            bn2 = gp["node_mlp"][1]["b"]

            ac, bc = _tc_pre(h, coord, w1[:, :D].T, b1[None, :],
                             w1[:, D:2 * D].T)
            pm, ps = [], []
            for hf in (0, 1):
                acr, bcc = _sc_gather(ac, bc, rowh[hf], colh[hf])
                ef, sm = _tc_edge(acr, bcc, eap[hf], rowe[hf],
                                  w1[:, 2 * D + 1:].T, w1[:, 2 * D][None, :],
                                  w2.T, b2[None, :], w3.T, b3[None, :], w4,
                                  elim[hf])
                pm.append(_sc_scatter_n(ef, row2[hf], zm))
                ps.append(_sc_scatter_8(sm, row82[hf], zm))
            h, coord = _tc_node(h, coord, pm[0], pm[1], ps[0], ps[1],
                                wn1[:, :D].T, wn1[:, D:].T, bn1[None, :],
                                wn2.T, bn2[None, :])
        wo = bp["emb_out"]
        h = _tc_linear(h, wo["W"].T, wo["b"][None, :])

    mol = _tc_mean(h)
    return (mol, h, coord[:, :3])
